# Initial kernel scaffold; baseline (speedup 1.0000x reference)
import jax
import jax.numpy as jnp
from jax.experimental import pallas as pl


def kernel(h, edge_index, W, A):
    raise NotImplementedError("write your pallas kernel here")


# speedup vs baseline: 12.8248x; 12.8248x over previous
import functools

import jax
import jax.numpy as jnp
from jax import lax
from jax.experimental import pallas as pl
from jax.experimental.pallas import tpu as pltpu
from jax.experimental.pallas import tpu_sc as plsc

N = 10000
E = 320000
IN_DIM = 128
OUT_DIM = 64
HEADS = 4

ACCW = 80
CH = 64
NT = 16
EPT = E // NT
FULL = EPT // CH
REM = EPT - FULL * CH
HPC = HEADS // 2
RPT = HPC * N // NT
RB = 50
BN = 1000
NB = N // BN


def _prep_body(h_ref, w_ref, a_ref, z_ref, s_ref, t_ref, sm_ref, tm_ref):
    @pl.when(pl.program_id(0) == 0)
    def _():
        sm_ref[...] = jnp.full((HEADS, 128), -jnp.inf, jnp.float32)
        tm_ref[...] = jnp.full((HEADS, 128), -jnp.inf, jnp.float32)

    hb = h_ref[...]
    sms, tms = [], []
    for i in range(HEADS):
        z = lax.dot_general(hb, w_ref[i], (((1,), (0,)), ((), ())),
                            preferred_element_type=jnp.float32)
        z_ref[i] = z
        sv = jnp.sum(z * a_ref[i, :OUT_DIM][None, :], axis=1)
        tv = jnp.sum(z * a_ref[i, OUT_DIM:][None, :], axis=1)
        s_ref[i, 0, 0, :] = sv
        t_ref[i, 0, 0, :] = tv
        sms.append(jnp.max(sv))
        tms.append(jnp.max(tv))
    smb = jnp.broadcast_to(jnp.stack(sms)[:, None], (HEADS, 128))
    tmb = jnp.broadcast_to(jnp.stack(tms)[:, None], (HEADS, 128))
    sm_ref[...] = jnp.maximum(sm_ref[...], smb)
    tm_ref[...] = jnp.maximum(tm_ref[...], tmb)


def _prep(h, W, A):
    return pl.pallas_call(
        _prep_body,
        grid=(N // BN,),
        in_specs=[
            pl.BlockSpec((BN, IN_DIM), lambda n: (n, 0)),
            pl.BlockSpec((HEADS, IN_DIM, OUT_DIM), lambda n: (0, 0, 0)),
            pl.BlockSpec((HEADS, 2 * OUT_DIM), lambda n: (0, 0)),
        ],
        out_specs=[
            pl.BlockSpec((HEADS, BN, OUT_DIM), lambda n: (0, n, 0)),
            pl.BlockSpec((HEADS, 1, 1, BN), lambda n: (0, n, 0, 0)),
            pl.BlockSpec((HEADS, 1, 1, BN), lambda n: (0, n, 0, 0)),
            pl.BlockSpec((HEADS, 128), lambda n: (0, 0)),
            pl.BlockSpec((HEADS, 128), lambda n: (0, 0)),
        ],
        out_shape=[
            jax.ShapeDtypeStruct((HEADS, N, OUT_DIM), jnp.float32),
            jax.ShapeDtypeStruct((HEADS, NB, 1, BN), jnp.float32),
            jax.ShapeDtypeStruct((HEADS, NB, 1, BN), jnp.float32),
            jax.ShapeDtypeStruct((HEADS, 128), jnp.float32),
            jax.ShapeDtypeStruct((HEADS, 128), jnp.float32),
        ],
    )(h, W, A)


def _sc_body(z_hbm, s_hbm, t_hbm, sm_hbm, tm_hbm, src_hbm, dst_hbm, out_hbm,
             s_loc, t_loc, gat, scat, wbuf, cbuf,
             ixs, ixd, ixg, ixs2, ixd2, ixg2, acc, sem):
    c = lax.axis_index("c")
    sid = lax.axis_index("s")
    zeros16 = jnp.zeros((16,), jnp.float32)

    def zrow(r, _):
        for j in range(ACCW // 16):
            scat[r, pl.ds(j * 16, 16)] = zeros16
        return 0
    lax.fori_loop(0, CH, zrow, 0)
    for b in range(RPT // RB):
        pltpu.sync_copy(scat.at[pl.ds(0, RB)],
                        acc.at[pl.ds(sid * RPT + b * RB, RB)])
    plsc.subcore_barrier()

    col_w = jnp.full((16,), OUT_DIM, jnp.int32)
    lanes = lax.iota(jnp.int32, 16)
    base0 = sid * EPT

    for ih in range(HPC):
        head = c * HPC + ih
        hoff = pl.multiple_of(head * N, 8)
        pltpu.sync_copy(s_hbm.at[pl.ds(hoff, N)], s_loc)
        pltpu.sync_copy(t_hbm.at[pl.ds(hoff, N)], t_loc)
        moff = pl.multiple_of(head * 128, 8)
        pltpu.sync_copy(sm_hbm.at[pl.ds(moff, 16)], cbuf)
        smax = cbuf[...]
        pltpu.sync_copy(tm_hbm.at[pl.ds(moff, 16)], cbuf)
        tmax = cbuf[...]
        cs = smax + tmax
        cshift = jnp.maximum(cs, cs * 0.01)
        zoff = head * N
        doff = ih * N

        def do_chunk(base, csize, xs, xd, xg):
            base = pl.multiple_of(base, 8)
            pltpu.sync_copy(src_hbm.at[pl.ds(base, csize)], xs)
            pltpu.sync_copy(dst_hbm.at[pl.ds(base, csize)], xd)
            for g in range(csize // 16):
                iv = xs[pl.ds(g * 16, 16)]
                jv = xd[pl.ds(g * 16, 16)]
                sv = plsc.load_gather(s_loc, [iv])
                tv = plsc.load_gather(t_loc, [jv])
                v = sv + tv
                v = jnp.maximum(v, v * 0.01)
                w = jnp.exp(v - cshift)
                wbuf[pl.ds(g * 16, 16)] = w
                xg[pl.ds(g * 16, 16)] = iv + zoff
                xd[pl.ds(g * 16, 16)] = jv + doff
                plsc.store_scatter(scat, [g * 16 + lanes, col_w], w)
            gdst = gat if csize == CH else gat.at[pl.ds(0, csize)]
            pltpu.async_copy(z_hbm.at[xg], gdst, sem).wait()

            def scale(e, _):
                ev = jnp.broadcast_to(e, (16,)).astype(jnp.int32)
                wv = plsc.load_gather(wbuf, [ev])
                for j in range(OUT_DIM // 16):
                    scat[e, pl.ds(j * 16, 16)] = gat[e, pl.ds(j * 16, 16)] * wv
                return 0
            lax.fori_loop(0, csize, scale, 0)
            ssrc = scat if csize == CH else scat.at[pl.ds(0, csize)]
            pltpu.sync_copy(ssrc, acc.at[xd], add=True)

        def chunk_body(k, _):
            do_chunk(base0 + k * CH, CH, ixs, ixd, ixg)
            return 0
        lax.fori_loop(0, FULL, chunk_body, 0)
        do_chunk(base0 + FULL * CH, REM, ixs2, ixd2, ixg2)

    plsc.subcore_barrier()

    head_mine = c * HPC + sid // (NT // HPC)
    node0 = (sid % (NT // HPC)) * RPT
    for b in range(RPT // RB):
        pltpu.sync_copy(acc.at[pl.ds(sid * RPT + b * RB, RB)],
                        scat.at[pl.ds(0, RB)])

        def drow(e, _):
            ev = jnp.broadcast_to(e, (16,)).astype(jnp.int32)
            dv = plsc.load_gather(scat, [ev, col_w])
            rec = 1.0 / (dv + 1e-16)
            for j in range(OUT_DIM // 16):
                gat[e, pl.ds(j * 16, 16)] = scat[e, pl.ds(j * 16, 16)] * rec
            return 0
        lax.fori_loop(0, RB, drow, 0)
        pltpu.sync_copy(
            gat.at[pl.ds(0, RB)],
            out_hbm.at[pl.ds(node0 + b * RB, RB),
                       pl.ds(head_mine * OUT_DIM, OUT_DIM)])


_gat_sc = functools.partial(
    pl.kernel,
    mesh=plsc.VectorSubcoreMesh(core_axis_name="c", subcore_axis_name="s"),
    compiler_params=pltpu.CompilerParams(needs_layout_passes=False,
                                         use_tc_tiling_on_sc=False),
    out_type=jax.ShapeDtypeStruct((N, HEADS * OUT_DIM), jnp.float32),
    scratch_types=[
        pltpu.VMEM((N,), jnp.float32),
        pltpu.VMEM((N,), jnp.float32),
        pltpu.VMEM((CH, OUT_DIM), jnp.float32),
        pltpu.VMEM((CH, ACCW), jnp.float32),
        pltpu.VMEM((CH,), jnp.float32),
        pltpu.VMEM((16,), jnp.float32),
        pltpu.VMEM((CH,), jnp.int32),
        pltpu.VMEM((CH,), jnp.int32),
        pltpu.VMEM((CH,), jnp.int32),
        pltpu.VMEM((REM,), jnp.int32),
        pltpu.VMEM((REM,), jnp.int32),
        pltpu.VMEM((REM,), jnp.int32),
        pltpu.VMEM_SHARED((HPC * N, ACCW), jnp.float32),
        pltpu.SemaphoreType.DMA,
    ],
)(_sc_body)


def kernel(h, edge_index, W, A):
    z, s, t, sm, tm = _prep(h, W, A)
    z_flat = z.reshape(HEADS * N, OUT_DIM)
    s = s.reshape(HEADS * N)
    t = t.reshape(HEADS * N)
    sm = sm.reshape(HEADS * 128)
    tm = tm.reshape(HEADS * 128)
    return _gat_sc(z_flat, s, t, sm, tm, edge_index[0], edge_index[1])


# speedup vs baseline: 40.4065x; 3.1506x over previous
import functools

import jax
import jax.numpy as jnp
from jax import lax
from jax.experimental import pallas as pl
from jax.experimental.pallas import tpu as pltpu
from jax.experimental.pallas import tpu_sc as plsc

N = 10000
E = 320000
IN_DIM = 128
OUT_DIM = 64
HEADS = 4

ACCW = 80
SUB = 64
NSLOT = 4
BODY = NSLOT * SUB
NT = 16
EPT = E // NT
NBODY = EPT // BODY
REM = EPT - NBODY * BODY
HPC = HEADS // 2
RPT = HPC * N // NT
RB = 50
BN = 1000
NB = N // BN


def _prep_body(h_ref, w_ref, a_ref, zaug_ref, t_ref, sm_ref, tm_ref):
    @pl.when(pl.program_id(0) == 0)
    def _():
        sm_ref[...] = jnp.full((HEADS, 128), -jnp.inf, jnp.float32)
        tm_ref[...] = jnp.full((HEADS, 128), -jnp.inf, jnp.float32)

    hb = h_ref[...]
    pad = jnp.zeros((BN, ACCW - OUT_DIM - 1), jnp.float32)
    sms, tms = [], []
    for i in range(HEADS):
        z = lax.dot_general(hb, w_ref[i], (((1,), (0,)), ((), ())),
                            preferred_element_type=jnp.float32)
        sv = jnp.sum(z * a_ref[i, :OUT_DIM][None, :], axis=1)
        tv = jnp.sum(z * a_ref[i, OUT_DIM:][None, :], axis=1)
        zaug_ref[i] = jnp.concatenate([z, sv[:, None], pad], axis=1)
        t_ref[i, 0, 0, :] = tv
        sms.append(jnp.max(sv))
        tms.append(jnp.max(tv))
    smb = jnp.broadcast_to(jnp.stack(sms)[:, None], (HEADS, 128))
    tmb = jnp.broadcast_to(jnp.stack(tms)[:, None], (HEADS, 128))
    sm_ref[...] = jnp.maximum(sm_ref[...], smb)
    tm_ref[...] = jnp.maximum(tm_ref[...], tmb)


def _prep(h, W, A):
    return pl.pallas_call(
        _prep_body,
        grid=(NB,),
        in_specs=[
            pl.BlockSpec((BN, IN_DIM), lambda n: (n, 0)),
            pl.BlockSpec((HEADS, IN_DIM, OUT_DIM), lambda n: (0, 0, 0)),
            pl.BlockSpec((HEADS, 2 * OUT_DIM), lambda n: (0, 0)),
        ],
        out_specs=[
            pl.BlockSpec((HEADS, BN, ACCW), lambda n: (0, n, 0)),
            pl.BlockSpec((HEADS, 1, 1, BN), lambda n: (0, n, 0, 0)),
            pl.BlockSpec((HEADS, 128), lambda n: (0, 0)),
            pl.BlockSpec((HEADS, 128), lambda n: (0, 0)),
        ],
        out_shape=[
            jax.ShapeDtypeStruct((HEADS, N, ACCW), jnp.float32),
            jax.ShapeDtypeStruct((HEADS, NB, 1, BN), jnp.float32),
            jax.ShapeDtypeStruct((HEADS, 128), jnp.float32),
            jax.ShapeDtypeStruct((HEADS, 128), jnp.float32),
        ],
    )(h, W, A)


def _sc_body(zaug_hbm, t_hbm, sm_hbm, tm_hbm, src_hbm, dst_hbm, out_hbm,
             gat, tv, xs, xd, xga, xdt, xda, cbuf, obuf,
             gr, tvr, xsr, xdr, xgar, xdtr, xdar,
             semz, semt, semi, semj, semsc, acc):
    c = lax.axis_index("c")
    sid = lax.axis_index("s")
    zeros16 = jnp.zeros((16,), jnp.float32)
    col_w = jnp.full((16,), OUT_DIM, jnp.int32)
    lanes = lax.iota(jnp.int32, 16)
    base0 = sid * EPT
    NG = SUB // 16

    def zrow(r, _):
        for q in range(NSLOT):
            for j in range(ACCW // 16):
                gat[q][r, pl.ds(j * 16, 16)] = zeros16
        return 0
    lax.fori_loop(0, SUB, zrow, 0)
    nzb = RPT // SUB
    for b in range(nzb):
        pltpu.sync_copy(gat[0], acc.at[pl.ds(sid * RPT + b * SUB, SUB)])
    if RPT - nzb * SUB:
        pltpu.sync_copy(gat[0].at[pl.ds(0, RPT - nzb * SUB)],
                        acc.at[pl.ds(sid * RPT + nzb * SUB, RPT - nzb * SUB)])
    plsc.subcore_barrier()

    for q in range(NSLOT):
        for g in range(NG):
            xda[q][pl.ds(g * 16, 16)] = g * 16 + lanes
        pltpu.async_copy(gat[q], acc.at[xda[q]], semsc.at[q], add=True)

    def nxt_base(k, q):
        off = jnp.minimum(k * BODY + q * SUB, EPT - SUB)
        return pl.multiple_of(base0 + off, 8)

    for ih in range(HPC):
        head = c * HPC + ih
        zoff = head * N
        doff = ih * N
        moff = pl.multiple_of(head * 128, 8)
        pltpu.sync_copy(sm_hbm.at[pl.ds(moff, 16)], cbuf)
        smax = cbuf[...]
        pltpu.sync_copy(tm_hbm.at[pl.ds(moff, 16)], cbuf)
        cs = smax + cbuf[...]
        cshift = jnp.maximum(cs, cs * 0.01)

        for q in range(NSLOT):
            b0 = pl.multiple_of(base0 + q * SUB, 8)
            pltpu.sync_copy(src_hbm.at[pl.ds(b0, SUB)], xs[q])
            pltpu.sync_copy(dst_hbm.at[pl.ds(b0, SUB)], xd[q])
            for g in range(NG):
                ds = pl.ds(g * 16, 16)
                xga[q][ds] = xs[q][ds] + zoff
                xdt[q][ds] = xd[q][ds] + zoff

        def body(k, _):
            for q in range(NSLOT):
                pltpu.make_async_copy(gat[q], acc.at[xda[q]], semsc.at[q]).wait()
                pltpu.async_copy(zaug_hbm.at[xga[q]], gat[q], semz.at[q])
                pltpu.async_copy(t_hbm.at[xdt[q]], tv[q], semt.at[q])
            for q in range(NSLOT):
                for g in range(NG):
                    ds = pl.ds(g * 16, 16)
                    xda[q][ds] = xd[q][ds] + doff
            for q in range(NSLOT):
                bq = nxt_base(k + 1, q)
                pltpu.async_copy(src_hbm.at[pl.ds(bq, SUB)], xs[q], semi.at[q])
                pltpu.async_copy(dst_hbm.at[pl.ds(bq, SUB)], xd[q], semj.at[q])
            for q in range(NSLOT):
                pltpu.make_async_copy(zaug_hbm.at[xga[q]], gat[q], semz.at[q]).wait()
                pltpu.make_async_copy(t_hbm.at[xdt[q]], tv[q], semt.at[q]).wait()
                for g in range(NG):
                    ds = pl.ds(g * 16, 16)
                    rows = g * 16 + lanes
                    sv = plsc.load_gather(gat[q], [rows, col_w])
                    v = sv + tv[q][ds]
                    v = jnp.maximum(v, v * 0.01)
                    w = jnp.exp(v - cshift)
                    plsc.store_scatter(gat[q], [rows, col_w], w)

                def scale(e, _):
                    ev = jnp.broadcast_to(e, (16,)).astype(jnp.int32)
                    wv = plsc.load_gather(gat[q], [ev, col_w])
                    for j in range(OUT_DIM // 16):
                        dsj = pl.ds(j * 16, 16)
                        gat[q][e, dsj] = gat[q][e, dsj] * wv
                    return 0
                lax.fori_loop(0, SUB, scale, 0)
                pltpu.async_copy(gat[q], acc.at[xda[q]], semsc.at[q], add=True)
            for q in range(NSLOT):
                bq = nxt_base(k + 1, q)
                pltpu.make_async_copy(src_hbm.at[pl.ds(bq, SUB)], xs[q],
                                      semi.at[q]).wait()
                pltpu.make_async_copy(dst_hbm.at[pl.ds(bq, SUB)], xd[q],
                                      semj.at[q]).wait()
                for g in range(NG):
                    ds = pl.ds(g * 16, 16)
                    xga[q][ds] = xs[q][ds] + zoff
                    xdt[q][ds] = xd[q][ds] + zoff
            return 0
        lax.fori_loop(0, NBODY, body, 0)

        if REM:
            br = pl.multiple_of(base0 + NBODY * BODY, 8)
            pltpu.sync_copy(src_hbm.at[pl.ds(br, REM)], xsr)
            pltpu.sync_copy(dst_hbm.at[pl.ds(br, REM)], xdr)
            for g in range(REM // 16):
                ds = pl.ds(g * 16, 16)
                xgar[ds] = xsr[ds] + zoff
                xdtr[ds] = xdr[ds] + zoff
                xdar[ds] = xdr[ds] + doff
            pltpu.sync_copy(zaug_hbm.at[xgar], gr)
            pltpu.sync_copy(t_hbm.at[xdtr], tvr)
            for g in range(REM // 16):
                ds = pl.ds(g * 16, 16)
                rows = g * 16 + lanes
                sv = plsc.load_gather(gr, [rows, col_w])
                v = sv + tvr[ds]
                v = jnp.maximum(v, v * 0.01)
                w = jnp.exp(v - cshift)
                plsc.store_scatter(gr, [rows, col_w], w)

            def scale_r(e, _):
                ev = jnp.broadcast_to(e, (16,)).astype(jnp.int32)
                wv = plsc.load_gather(gr, [ev, col_w])
                for j in range(OUT_DIM // 16):
                    dsj = pl.ds(j * 16, 16)
                    gr[e, dsj] = gr[e, dsj] * wv
                return 0
            lax.fori_loop(0, REM, scale_r, 0)
            pltpu.sync_copy(gr, acc.at[xdar], add=True)

    for q in range(NSLOT):
        pltpu.make_async_copy(gat[q], acc.at[xda[q]], semsc.at[q]).wait()
    plsc.subcore_barrier()

    head_mine = c * HPC + sid // (NT // HPC)
    node0 = (sid % (NT // HPC)) * RPT
    for b in range(RPT // RB):
        pltpu.sync_copy(acc.at[pl.ds(sid * RPT + b * RB, RB)],
                        gat[0].at[pl.ds(0, RB)])

        def drow(e, _):
            ev = jnp.broadcast_to(e, (16,)).astype(jnp.int32)
            dv = plsc.load_gather(gat[0], [ev, col_w])
            rec = 1.0 / (dv + 1e-16)
            for j in range(OUT_DIM // 16):
                dsj = pl.ds(j * 16, 16)
                obuf[e, dsj] = gat[0][e, dsj] * rec
            return 0
        lax.fori_loop(0, RB, drow, 0)
        pltpu.sync_copy(
            obuf.at[pl.ds(0, RB)],
            out_hbm.at[pl.ds(node0 + b * RB, RB),
                       pl.ds(head_mine * OUT_DIM, OUT_DIM)])


_gat_sc = functools.partial(
    pl.kernel,
    mesh=plsc.VectorSubcoreMesh(core_axis_name="c", subcore_axis_name="s"),
    compiler_params=pltpu.CompilerParams(needs_layout_passes=False,
                                         use_tc_tiling_on_sc=False),
    out_type=jax.ShapeDtypeStruct((N, HEADS * OUT_DIM), jnp.float32),
    scratch_types=[
        [pltpu.VMEM((SUB, ACCW), jnp.float32) for _ in range(NSLOT)],
        [pltpu.VMEM((SUB,), jnp.float32) for _ in range(NSLOT)],
        [pltpu.VMEM((SUB,), jnp.int32) for _ in range(NSLOT)],
        [pltpu.VMEM((SUB,), jnp.int32) for _ in range(NSLOT)],
        [pltpu.VMEM((SUB,), jnp.int32) for _ in range(NSLOT)],
        [pltpu.VMEM((SUB,), jnp.int32) for _ in range(NSLOT)],
        [pltpu.VMEM((SUB,), jnp.int32) for _ in range(NSLOT)],
        pltpu.VMEM((16,), jnp.float32),
        pltpu.VMEM((RB, OUT_DIM), jnp.float32),
        pltpu.VMEM((REM, ACCW), jnp.float32),
        pltpu.VMEM((REM,), jnp.float32),
        pltpu.VMEM((REM,), jnp.int32),
        pltpu.VMEM((REM,), jnp.int32),
        pltpu.VMEM((REM,), jnp.int32),
        pltpu.VMEM((REM,), jnp.int32),
        pltpu.VMEM((REM,), jnp.int32),
        pltpu.SemaphoreType.DMA((NSLOT,)),
        pltpu.SemaphoreType.DMA((NSLOT,)),
        pltpu.SemaphoreType.DMA((NSLOT,)),
        pltpu.SemaphoreType.DMA((NSLOT,)),
        pltpu.SemaphoreType.DMA((NSLOT,)),
        pltpu.VMEM_SHARED((HPC * N, ACCW), jnp.float32),
    ],
)(_sc_body)


def kernel(h, edge_index, W, A):
    zaug, t, sm, tm = _prep(h, W, A)
    zaug_flat = zaug.reshape(HEADS * N, ACCW)
    t_flat = t.reshape(HEADS * N)
    sm = sm.reshape(HEADS * 128)
    tm = tm.reshape(HEADS * 128)
    return _gat_sc(zaug_flat, t_flat, sm, tm, edge_index[0], edge_index[1])
